# baseline (speedup 1.0000x reference)
import functools

import jax
import jax.numpy as jnp
from jax import lax
from jax.experimental import pallas as pl
from jax.experimental.pallas import tpu as pltpu
from jax.experimental.pallas import tpu_sc as plsc

_OOV = 100
_LANES = 16


def _final_dist_sc(B, V, L):
    E = V + _OOV
    EP = ((E + _LANES - 1) // _LANES) * _LANES
    LP = ((L + _LANES - 1) // _LANES) * _LANES
    assert V % _LANES == 0

    info = plsc.get_sparse_core_info()
    NW = info.num_cores * info.num_subcores
    assert B % NW == 0
    R = B // NW

    mesh = plsc.VectorSubcoreMesh(core_axis_name="c", subcore_axis_name="s")

    @functools.partial(
        pl.kernel,
        mesh=mesh,
        compiler_params=pltpu.CompilerParams(
            needs_layout_passes=False, use_tc_tiling_on_sc=False),
        out_type=jax.ShapeDtypeStruct((B, E), jnp.float32),
        scratch_types=[
            pltpu.VMEM((EP,), jnp.float32),
            pltpu.VMEM((LP,), jnp.float32),
            pltpu.VMEM((LP,), jnp.int32),
            pltpu.VMEM((_LANES,), jnp.float32),
        ],
    )
    def sc_fn(vocab_hbm, attn_hbm, ids_hbm, pg_hbm, out_hbm, buf, attnv, idsv, pgv):
        wid = lax.axis_index("s") * info.num_cores + lax.axis_index("c")

        def row_body(k, carry):
            b = wid * R + k
            pltpu.sync_copy(vocab_hbm.at[b], buf.at[pl.ds(0, V)])
            pltpu.sync_copy(pg_hbm.at[b], pgv)
            pltpu.sync_copy(attn_hbm.at[b], attnv)
            pltpu.sync_copy(ids_hbm.at[b], idsv)
            for i in range((EP - V) // _LANES):
                buf[pl.ds(V + _LANES * i, _LANES)] = jnp.zeros(
                    (_LANES,), jnp.float32)
            pg = pgv[...]
            one_minus_pg = 1.0 - pg

            def scale_body(i, c):
                sl = pl.ds(i * _LANES, _LANES)
                buf[sl] = buf[sl] * pg
                return c

            lax.fori_loop(0, V // _LANES, scale_body, None)

            for j in range(LP // _LANES):
                sl = pl.ds(j * _LANES, _LANES)
                idx = idsv[sl]
                val = attnv[sl] * one_minus_pg
                mask = (idx >= 0) & (idx < E)
                plsc.addupdate_scatter(buf, [idx], val, mask=mask)

            pltpu.sync_copy(buf.at[pl.ds(0, E)], out_hbm.at[b])
            return carry

        lax.fori_loop(0, R, row_body, None)

    return sc_fn


def kernel(vocab_dists, attn_dists, p_gens, input_ids):
    T, B, V = vocab_dists.shape
    L = attn_dists.shape[-1]
    LP = ((L + _LANES - 1) // _LANES) * _LANES

    sc_fn = _final_dist_sc(B, V, L)

    outs = []
    for t in range(T):
        vocab = vocab_dists[t]
        pg_rep = jnp.broadcast_to(p_gens[t], (B, _LANES))
        attn_pad = jnp.pad(attn_dists[t], ((0, 0), (0, LP - L)))
        ids_pad = jnp.pad(input_ids, ((0, 0), (0, LP - L)),
                          constant_values=-1)
        outs.append(sc_fn(vocab, attn_pad, ids_pad, pg_rep))
    return jnp.stack(outs, axis=0)


# speedup vs baseline: 5.9084x; 5.9084x over previous
import functools

import jax
import jax.numpy as jnp
from jax import lax
from jax.experimental import pallas as pl
from jax.experimental.pallas import tpu as pltpu
from jax.experimental.pallas import tpu_sc as plsc

_OOV = 100
_LANES = 16
_TILE = 128
_SUB = 8


def _final_dist_sc_tiled(B, V, L):
    E = V + _OOV
    ET = (E // _TILE) * _TILE
    TAIL = E - ET
    TAILP = _LANES
    NT = ET // _TILE
    NVT = V // _TILE
    TB = 112
    NBLK = -(-NT // TB)
    LP = ((L + _LANES - 1) // _LANES) * _LANES
    VTW = V - NVT * _TILE

    info = plsc.get_sparse_core_info()
    NW = info.num_cores * info.num_subcores
    NS = B // _SUB
    SPW = NS // NW

    mesh = plsc.VectorSubcoreMesh(core_axis_name="c", subcore_axis_name="s")

    @functools.partial(
        pl.kernel,
        mesh=mesh,
        compiler_params=pltpu.CompilerParams(needs_layout_passes=False),
        out_type=(
            jax.ShapeDtypeStruct((B, E), jnp.float32),
            jax.ShapeDtypeStruct((B * TAILP,), jnp.float32),
        ),
        scratch_types=[
            pltpu.VMEM((TB, _SUB, _TILE), jnp.float32),
            pltpu.VMEM((_SUB * TAILP,), jnp.float32),
            pltpu.VMEM((_SUB * LP,), jnp.float32),
            pltpu.VMEM((_SUB * LP,), jnp.int32),
            pltpu.VMEM((_SUB * _LANES,), jnp.float32),
            pltpu.VMEM((_SUB * VTW,), jnp.float32),
            pltpu.SemaphoreType.DMA,
        ],
    )
    def sc_fn(vocab_hbm, attn_hbm, ids_hbm, pg_hbm, vt_hbm,
              out_hbm, tail_hbm, buf, tailb, attnb, idsb, pgb, vtb, sem):
        wid = lax.axis_index("s") * info.num_cores + lax.axis_index("c")

        def stripe_body(si, carry):
            s = wid * SPW + si
            r0 = s * _SUB
            pltpu.sync_copy(attn_hbm.at[pl.ds(r0 * LP, _SUB * LP)], attnb)
            pltpu.sync_copy(ids_hbm.at[pl.ds(r0 * LP, _SUB * LP)], idsb)
            pltpu.sync_copy(pg_hbm.at[pl.ds(r0 * _LANES, _SUB * _LANES)], pgb)
            pltpu.sync_copy(vt_hbm.at[pl.ds(r0 * VTW, _SUB * VTW)], vtb)

            pgs = [pgb[pl.ds(r * _LANES, _LANES)] for r in range(_SUB)]
            omps = [1.0 - p for p in pgs]

            for i in range(_SUB * TAILP // _LANES):
                tailb[pl.ds(i * _LANES, _LANES)] = jnp.zeros(
                    (_LANES,), jnp.float32)
            for r in range(_SUB):
                for j in range(LP // _LANES):
                    sl = pl.ds(r * LP + j * _LANES, _LANES)
                    idv = idsb[sl]
                    local = idv - ET
                    m = (local >= 0) & (local < TAIL)
                    val = attnb[sl] * omps[r]
                    plsc.addupdate_scatter(
                        tailb, [jnp.full((_LANES,), r * TAILP, jnp.int32)
                                + local], val, mask=m)
            pltpu.sync_copy(tailb, tail_hbm.at[pl.ds(r0 * TAILP, _SUB * TAILP)])

            def blk_body(blk, c2):
                k0 = blk * TB
                tb = jnp.minimum(TB, NT - k0)
                is_last = blk == NBLK - 1
                nin = tb - is_last.astype(jnp.int32)

                def dma_in(j, c3):
                    pltpu.make_async_copy(
                        vocab_hbm.at[pl.ds(r0, _SUB),
                                     pl.ds((k0 + j) * _TILE, _TILE)],
                        buf.at[j], sem).start()
                    return c3
                lax.fori_loop(0, nin, dma_in, None)

                def dma_in_wait(j, c3):
                    pltpu.make_async_copy(
                        vocab_hbm.at[pl.ds(r0, _SUB),
                                     pl.ds((k0 + j) * _TILE, _TILE)],
                        buf.at[j], sem).wait()
                    return c3
                lax.fori_loop(0, nin, dma_in_wait, None)

                @pl.when(is_last)
                def _():
                    jsp = tb - 1
                    zero = jnp.zeros((_LANES,), jnp.float32)
                    for r in range(_SUB):
                        for c in range(VTW // _LANES):
                            buf[jsp, r, pl.ds(c * _LANES, _LANES)] = (
                                vtb[pl.ds(r * VTW + c * _LANES, _LANES)])
                        for c in range(VTW // _LANES, _TILE // _LANES):
                            buf[jsp, r, pl.ds(c * _LANES, _LANES)] = zero

                def scale(j, c3):
                    for r in range(_SUB):
                        for c in range(_TILE // _LANES):
                            sl = pl.ds(c * _LANES, _LANES)
                            buf[j, r, sl] = buf[j, r, sl] * pgs[r]
                    return c3
                lax.fori_loop(0, tb, scale, None)

                lo = k0 * _TILE
                hi = lo + tb * _TILE
                for r in range(_SUB):
                    rvec = jnp.full((_LANES,), r, jnp.int32)
                    for j in range(LP // _LANES):
                        sl = pl.ds(r * LP + j * _LANES, _LANES)
                        idv = idsb[sl]
                        local = idv - lo
                        m = (idv >= lo) & (idv < hi)
                        tv = lax.shift_right_logical(local, 7)
                        lv = lax.bitwise_and(local, 127)
                        val = attnb[sl] * omps[r]
                        plsc.addupdate_scatter(
                            buf, [tv, rvec, lv], val, mask=m)

                def dma_out(j, c3):
                    pltpu.make_async_copy(
                        buf.at[j],
                        out_hbm.at[pl.ds(r0, _SUB),
                                   pl.ds((k0 + j) * _TILE, _TILE)],
                        sem).start()
                    return c3
                lax.fori_loop(0, tb, dma_out, None)

                def dma_out_wait(j, c3):
                    pltpu.make_async_copy(
                        buf.at[j],
                        out_hbm.at[pl.ds(r0, _SUB),
                                   pl.ds((k0 + j) * _TILE, _TILE)],
                        sem).wait()
                    return c3
                lax.fori_loop(0, tb, dma_out_wait, None)
                return c2

            lax.fori_loop(0, NBLK, blk_body, None)
            return carry

        lax.fori_loop(0, SPW, stripe_body, None)

    return sc_fn, ET, TAIL, TAILP


def _tail_merge_tc(B, E, ET):
    jlast = ET // _TILE

    def body(t_ref, o_in_ref, o_ref):
        o_ref[...] = t_ref[...]

    return pl.pallas_call(
        body,
        grid=(B // _SUB,),
        in_specs=[
            pl.BlockSpec((_SUB, _TILE), lambda i: (i, 0)),
            pl.BlockSpec((_SUB, _TILE), lambda i: (i, jlast)),
        ],
        out_specs=pl.BlockSpec((_SUB, _TILE), lambda i: (i, jlast)),
        out_shape=jax.ShapeDtypeStruct((B, E), jnp.float32),
        input_output_aliases={1: 0},
    )


def kernel(vocab_dists, attn_dists, p_gens, input_ids):
    T, B, V = vocab_dists.shape
    L = attn_dists.shape[-1]
    E = V + _OOV
    LP = ((L + _LANES - 1) // _LANES) * _LANES

    sc_fn, ET, TAIL, TAILP = _final_dist_sc_tiled(B, V, L)
    merge = _tail_merge_tc(B, E, ET)
    NVT = V // _TILE

    outs = []
    for t in range(T):
        vocab = vocab_dists[t]
        vt = vocab[:, NVT * _TILE:].reshape(-1)
        pg_flat = jnp.broadcast_to(
            p_gens[t], (B, _LANES)).reshape(-1)
        attn_flat = jnp.pad(
            attn_dists[t], ((0, 0), (0, LP - L))).reshape(-1)
        ids_flat = jnp.pad(
            input_ids, ((0, 0), (0, LP - L)),
            constant_values=-1).reshape(-1)
        out1, tail = sc_fn(vocab, attn_flat, ids_flat, pg_flat, vt)
        tail4 = jnp.pad(tail.reshape(B, TAILP)[:, :TAIL],
                        ((0, 0), (0, _TILE - TAIL)))
        outs.append(merge(tail4, out1))
    return jnp.stack(outs, axis=0)


# speedup vs baseline: 6.5925x; 1.1158x over previous
import functools

import jax
import jax.numpy as jnp
from jax import lax
from jax.experimental import pallas as pl
from jax.experimental.pallas import tpu as pltpu
from jax.experimental.pallas import tpu_sc as plsc

_OOV = 100
_LANES = 16
_TILE = 128
_SUB = 8


def _final_dist_sc_tiled(B, V, L):
    E = V + _OOV
    ET = (E // _TILE) * _TILE
    TAIL = E - ET
    TAILP = _LANES
    NT = ET // _TILE
    NVT = V // _TILE
    TB = 37
    NBLK = -(-NT // TB)
    LP = ((L + _LANES - 1) // _LANES) * _LANES
    VTW = V - NVT * _TILE

    info = plsc.get_sparse_core_info()
    NW = info.num_cores * info.num_subcores
    NS = B // _SUB
    SPW = NS // NW

    mesh = plsc.VectorSubcoreMesh(core_axis_name="c", subcore_axis_name="s")

    @functools.partial(
        pl.kernel,
        mesh=mesh,
        compiler_params=pltpu.CompilerParams(needs_layout_passes=False),
        out_type=(
            jax.ShapeDtypeStruct((B, E), jnp.float32),
            jax.ShapeDtypeStruct((B * TAILP,), jnp.float32),
        ),
        scratch_types=[
            pltpu.VMEM((TB, _SUB, _TILE), jnp.float32),
            pltpu.VMEM((TB, _SUB, _TILE), jnp.float32),
            pltpu.VMEM((TB, _SUB, _TILE), jnp.float32),
            pltpu.VMEM((_SUB * TAILP,), jnp.float32),
            pltpu.VMEM((_SUB * LP,), jnp.float32),
            pltpu.VMEM((_SUB * LP,), jnp.int32),
            pltpu.VMEM((_SUB * _LANES,), jnp.float32),
            pltpu.VMEM((_SUB * VTW,), jnp.float32),
            pltpu.SemaphoreType.DMA,
            pltpu.SemaphoreType.DMA,
            pltpu.SemaphoreType.DMA,
            pltpu.SemaphoreType.DMA,
            pltpu.SemaphoreType.DMA,
            pltpu.SemaphoreType.DMA,
        ],
    )
    def sc_fn(vocab_hbm, attn_hbm, ids_hbm, pg_hbm, vt_hbm,
              out_hbm, tail_hbm, buf0, buf1, buf2, tailb, attnb, idsb,
              pgb, vtb, si0, si1, si2, so0, so1, so2):
        wid = lax.axis_index("s") * info.num_cores + lax.axis_index("c")
        bufs = (buf0, buf1, buf2)
        isems = (si0, si1, si2)
        osems = (so0, so1, so2)

        def nin_of(blk):
            blk = jnp.asarray(blk, jnp.int32)
            tb = jnp.minimum(TB, NT - blk * TB)
            return tb - (blk == NBLK - 1).astype(jnp.int32)

        def stripe_body(si, carry):
            s = wid * SPW + si
            r0 = s * _SUB
            pltpu.sync_copy(attn_hbm.at[pl.ds(r0 * LP, _SUB * LP)], attnb)
            pltpu.sync_copy(ids_hbm.at[pl.ds(r0 * LP, _SUB * LP)], idsb)
            pltpu.sync_copy(pg_hbm.at[pl.ds(r0 * _LANES, _SUB * _LANES)], pgb)
            pltpu.sync_copy(vt_hbm.at[pl.ds(r0 * VTW, _SUB * VTW)], vtb)

            pgs = [pgb[pl.ds(r * _LANES, _LANES)] for r in range(_SUB)]

            for r in range(_SUB):
                omp = 1.0 - pgs[r]
                for j in range(LP // _LANES):
                    sl = pl.ds(r * LP + j * _LANES, _LANES)
                    attnb[sl] = attnb[sl] * omp

            for i in range(_SUB * TAILP // _LANES):
                tailb[pl.ds(i * _LANES, _LANES)] = jnp.zeros(
                    (_LANES,), jnp.float32)
            for r in range(_SUB):
                for j in range(LP // _LANES):
                    sl = pl.ds(r * LP + j * _LANES, _LANES)
                    idv = idsb[sl]
                    local = idv - ET
                    m = (local >= 0) & (local < TAIL)
                    plsc.addupdate_scatter(
                        tailb, [jnp.full((_LANES,), r * TAILP, jnp.int32)
                                + local], attnb[sl], mask=m)
            pltpu.sync_copy(tailb, tail_hbm.at[pl.ds(r0 * TAILP, _SUB * TAILP)])

            def fire_in(buf, sem, blk):
                def f(j, c):
                    pltpu.make_async_copy(
                        vocab_hbm.at[pl.ds(r0, _SUB),
                                     pl.ds((blk * TB + j) * _TILE, _TILE)],
                        buf.at[j], sem).start()
                    return c
                lax.fori_loop(0, nin_of(blk), f, None)

            def wait_in(buf, sem, blk):
                def f(j, c):
                    pltpu.make_async_copy(
                        vocab_hbm.at[pl.ds(r0, _SUB),
                                     pl.ds((blk * TB + j) * _TILE, _TILE)],
                        buf.at[j], sem).wait()
                    return c
                lax.fori_loop(0, nin_of(blk), f, None)

            def fire_out(buf, sem, blk):
                tb = jnp.minimum(TB, NT - blk * TB)

                def f(j, c):
                    pltpu.make_async_copy(
                        buf.at[j],
                        out_hbm.at[pl.ds(r0, _SUB),
                                   pl.ds((blk * TB + j) * _TILE, _TILE)],
                        sem).start()
                    return c
                lax.fori_loop(0, tb, f, None)

            def wait_out(buf, sem, blk):
                tb = jnp.minimum(TB, NT - blk * TB)

                def f(j, c):
                    pltpu.make_async_copy(
                        buf.at[j],
                        out_hbm.at[pl.ds(r0, _SUB),
                                   pl.ds((blk * TB + j) * _TILE, _TILE)],
                        sem).wait()
                    return c
                lax.fori_loop(0, tb, f, None)

            def process(buf, blk):
                tb = jnp.minimum(TB, NT - blk * TB)
                is_last = blk == NBLK - 1

                @pl.when(is_last)
                def _():
                    jsp = tb - 1
                    zero = jnp.zeros((_LANES,), jnp.float32)
                    for r in range(_SUB):
                        for c in range(VTW // _LANES):
                            buf[jsp, r, pl.ds(c * _LANES, _LANES)] = (
                                vtb[pl.ds(r * VTW + c * _LANES, _LANES)])
                        for c in range(VTW // _LANES, _TILE // _LANES):
                            buf[jsp, r, pl.ds(c * _LANES, _LANES)] = zero

                def scale(j, c3):
                    for r in range(_SUB):
                        for c in range(_TILE // _LANES):
                            sl = pl.ds(c * _LANES, _LANES)
                            buf[j, r, sl] = buf[j, r, sl] * pgs[r]
                    return c3
                lax.fori_loop(0, tb, scale, None)

                lo = blk * TB * _TILE
                hi = lo + tb * _TILE
                for r in range(_SUB):
                    rvec = jnp.full((_LANES,), r, jnp.int32)
                    for j in range(LP // _LANES):
                        sl = pl.ds(r * LP + j * _LANES, _LANES)
                        idv = idsb[sl]
                        local = idv - lo
                        m = (idv >= lo) & (idv < hi)
                        tv = lax.shift_right_logical(local, 7)
                        lv = lax.bitwise_and(local, 127)
                        plsc.addupdate_scatter(
                            buf, [tv, rvec, lv], attnb[sl], mask=m)

            fire_in(buf0, si0, 0)
            fire_in(buf1, si1, 1)

            def blk_step(g, c2):
                for b in range(3):
                    @pl.when(g % 3 == b)
                    def _():
                        wait_in(bufs[b], isems[b], g)
                        process(bufs[b], g)
                        fire_out(bufs[b], osems[b], g)
                        bp = (b + 2) % 3

                        @pl.when(g + 2 < NBLK)
                        def _():
                            @pl.when(g >= 1)
                            def _():
                                wait_out(bufs[bp], osems[bp], g - 1)
                            fire_in(bufs[bp], isems[bp], g + 2)
                return c2

            lax.fori_loop(0, NBLK, blk_step, None)
            for blk in (NBLK - 3, NBLK - 2, NBLK - 1):
                wait_out(bufs[blk % 3], osems[blk % 3], blk)
            return carry

        lax.fori_loop(0, SPW, stripe_body, None)

    return sc_fn, ET, TAIL, TAILP


def _tail_merge_tc(B, E, ET):
    jlast = ET // _TILE

    def body(t_ref, o_in_ref, o_ref):
        o_ref[...] = t_ref[...]

    return pl.pallas_call(
        body,
        grid=(B // _SUB,),
        in_specs=[
            pl.BlockSpec((_SUB, _TILE), lambda i: (i, 0)),
            pl.BlockSpec((_SUB, _TILE), lambda i: (i, jlast)),
        ],
        out_specs=pl.BlockSpec((_SUB, _TILE), lambda i: (i, jlast)),
        out_shape=jax.ShapeDtypeStruct((B, E), jnp.float32),
        input_output_aliases={1: 0},
    )


def kernel(vocab_dists, attn_dists, p_gens, input_ids):
    T, B, V = vocab_dists.shape
    L = attn_dists.shape[-1]
    E = V + _OOV
    LP = ((L + _LANES - 1) // _LANES) * _LANES

    sc_fn, ET, TAIL, TAILP = _final_dist_sc_tiled(B, V, L)
    merge = _tail_merge_tc(B, E, ET)
    NVT = V // _TILE

    outs = []
    for t in range(T):
        vocab = vocab_dists[t]
        vt = vocab[:, NVT * _TILE:].reshape(-1)
        pg_flat = jnp.broadcast_to(
            p_gens[t], (B, _LANES)).reshape(-1)
        attn_flat = jnp.pad(
            attn_dists[t], ((0, 0), (0, LP - L))).reshape(-1)
        ids_flat = jnp.pad(
            input_ids, ((0, 0), (0, LP - L)),
            constant_values=-1).reshape(-1)
        out1, tail = sc_fn(vocab, attn_flat, ids_flat, pg_flat, vt)
        tail4 = jnp.pad(tail.reshape(B, TAILP)[:, :TAIL],
                        ((0, 0), (0, _TILE - TAIL)))
        outs.append(merge(tail4, out1))
    return jnp.stack(outs, axis=0)
